# baseline (speedup 1.0000x reference)
import functools

import jax
import jax.numpy as jnp
import numpy as np
from jax import lax
from jax.experimental import pallas as pl
from jax.experimental.pallas import tpu as pltpu
from jax.experimental.pallas import tpu_sc as plsc

VOCAB = 100000
MIN_N = 2
MAX_N = 3
NUM_HEADS = 4
TBL = 100000
EDIM = 64
HID = 2048
KSIZE = 3
ORDERS = MAX_N - MIN_N + 1
MEMD = ORDERS * NUM_HEADS * EDIM
BATCH = 2
SEQ = 4096
BS = BATCH * SEQ

_rng = np.random.RandomState(1234)
_HASH_MULT = ((_rng.randint(1, 2**31 - 1, size=(NUM_HEADS, MAX_N)) * 2 + 1)
              % (2**32)).astype(np.uint32)

NW = 32
CHUNK = BS // NW
GSUB = 128
NSUB = CHUNK // GSUB


def _sc_gather_mem(tables_flat, idt, idm1, idm2):
    mesh = plsc.VectorSubcoreMesh(core_axis_name="c", subcore_axis_name="s")

    @functools.partial(
        pl.kernel,
        mesh=mesh,
        compiler_params=pltpu.CompilerParams(use_tc_tiling_on_sc=False),
        out_type=jax.ShapeDtypeStruct((ORDERS * NUM_HEADS, BS, EDIM), jnp.float32),
        scratch_types=[
            pltpu.VMEM((CHUNK,), jnp.int32),
            pltpu.VMEM((CHUNK,), jnp.int32),
            pltpu.VMEM((CHUNK,), jnp.int32),
            pltpu.VMEM((NSUB, GSUB), jnp.int32),
            pltpu.VMEM((CHUNK, EDIM), jnp.float32),
            pltpu.SemaphoreType.DMA,
        ],
    )
    def k(tbl_hbm, idt_hbm, idm1_hbm, idm2_hbm, out_hbm,
          idt_v, idm1_v, idm2_v, idx_v, rows_v, sem):
        wid = lax.axis_index("s") * 2 + lax.axis_index("c")
        base = wid * CHUNK
        pltpu.sync_copy(idt_hbm.at[pl.ds(base, CHUNK)], idt_v)
        pltpu.sync_copy(idm1_hbm.at[pl.ds(base, CHUNK)], idm1_v)
        pltpu.sync_copy(idm2_hbm.at[pl.ds(base, CHUNK)], idm2_v)

        for o, n in enumerate(range(MIN_N, MAX_N + 1)):
            for h in range(NUM_HEADS):
                kk = o * NUM_HEADS + h
                m = _HASH_MULT[h]
                for v in range(CHUNK // 16):
                    sl = pl.ds(v * 16, 16)
                    t0 = plsc.bitcast(idt_v[sl], jnp.uint32)
                    t1 = plsc.bitcast(idm1_v[sl], jnp.uint32)
                    if n == 2:
                        acc = t1 * jnp.uint32(m[0]) + t0 * jnp.uint32(m[1])
                    else:
                        t2 = plsc.bitcast(idm2_v[sl], jnp.uint32)
                        acc = (t2 * jnp.uint32(m[0]) + t1 * jnp.uint32(m[1])
                               + t0 * jnp.uint32(m[2]))
                    acc = acc ^ (acc >> jnp.uint32(16))
                    acc = acc % jnp.uint32(TBL)
                    row = plsc.bitcast(acc, jnp.int32) + jnp.int32(kk * TBL)
                    idx_v[v // (GSUB // 16), pl.ds((v % (GSUB // 16)) * 16, 16)] = row
                cps = [
                    pltpu.async_copy(
                        tbl_hbm.at[idx_v.at[c]],
                        rows_v.at[pl.ds(c * GSUB, GSUB)],
                        sem,
                    )
                    for c in range(NSUB)
                ]
                for cp in cps:
                    cp.wait()
                pltpu.sync_copy(rows_v, out_hbm.at[kk, pl.ds(base, CHUNK)])

    return k(tables_flat, idt, idm1, idm2)


_TBLK = 512


def _tc_body(hs_ref, mem_ref, wg_ref, wv_ref, bg_ref, bv_ref, cw_ref, cb_ref,
             out_ref, carry_ref):
    j = pl.program_id(1)
    hs = hs_ref[0]
    mem = jnp.concatenate(
        [mem_ref[kk, 0] for kk in range(ORDERS * NUM_HEADS)], axis=1
    )
    memh = mem.astype(jnp.bfloat16)
    keyv = jnp.dot(hs.astype(jnp.bfloat16), wg_ref[...],
                   preferred_element_type=jnp.float32) + bg_ref[...]
    dot = jnp.sum(keyv * mem, axis=1, keepdims=True) * (1.0 / np.sqrt(MEMD))
    alpha = 1.0 / (1.0 + jnp.exp(-dot))
    value = jnp.dot(memh, wv_ref[...],
                    preferred_element_type=jnp.float32) + bv_ref[...]
    gated = alpha * value
    prev = jnp.where(j == 0, 0.0, carry_ref[0:2])
    g_m1 = jnp.concatenate([prev[1:2], gated[:-1]], axis=0)
    g_m2 = jnp.concatenate([prev[0:2], gated[:-2]], axis=0)
    fused = (g_m2 * cw_ref[0:1] + g_m1 * cw_ref[1:2] + gated * cw_ref[2:3]
             + cb_ref[...])
    out_ref[0] = hs + fused
    carry_ref[0:2] = gated[_TBLK - 2:]


def _tc_dense(hs, mem3, Wg, bg, Wv, bv, conv_w, conv_b):
    grid = (BATCH, SEQ // _TBLK)
    return pl.pallas_call(
        _tc_body,
        grid=grid,
        in_specs=[
            pl.BlockSpec((1, _TBLK, HID), lambda b, j: (b, j, 0)),
            pl.BlockSpec((ORDERS * NUM_HEADS, 1, _TBLK, EDIM),
                         lambda b, j: (0, b, j, 0)),
            pl.BlockSpec((HID, MEMD), lambda b, j: (0, 0)),
            pl.BlockSpec((MEMD, HID), lambda b, j: (0, 0)),
            pl.BlockSpec((1, MEMD), lambda b, j: (0, 0)),
            pl.BlockSpec((1, HID), lambda b, j: (0, 0)),
            pl.BlockSpec((KSIZE, HID), lambda b, j: (0, 0)),
            pl.BlockSpec((1, HID), lambda b, j: (0, 0)),
        ],
        out_specs=pl.BlockSpec((1, _TBLK, HID), lambda b, j: (b, j, 0)),
        out_shape=jax.ShapeDtypeStruct((BATCH, SEQ, HID), jnp.float32),
        scratch_shapes=[pltpu.VMEM((8, HID), jnp.float32)],
        compiler_params=pltpu.CompilerParams(
            dimension_semantics=("arbitrary", "arbitrary"),
        ),
    )(hs, mem3, Wg.astype(jnp.bfloat16), Wv.astype(jnp.bfloat16),
      bg.reshape(1, MEMD), bv.reshape(1, HID),
      conv_w.T, conv_b.reshape(1, HID))


def kernel(hidden_states, input_ids, tables, Wg, bg, Wv, bv, conv_w, conv_b):
    ids = input_ids.astype(jnp.int32)
    idm1 = jnp.pad(ids, ((0, 0), (1, 0)))[:, :SEQ]
    idm2 = jnp.pad(ids, ((0, 0), (2, 0)))[:, :SEQ]
    tables_flat = tables.reshape(ORDERS * NUM_HEADS * TBL, EDIM)
    mem = _sc_gather_mem(tables_flat, ids.reshape(BS), idm1.reshape(BS),
                         idm2.reshape(BS))
    mem4 = mem.reshape(ORDERS * NUM_HEADS, BATCH, SEQ, EDIM)
    return _tc_dense(hidden_states, mem4, Wg, bg, Wv, bv, conv_w, conv_b)


# speedup vs baseline: 1.0012x; 1.0012x over previous
import functools

import jax
import jax.numpy as jnp
import numpy as np
from jax import lax
from jax.experimental import pallas as pl
from jax.experimental.pallas import tpu as pltpu
from jax.experimental.pallas import tpu_sc as plsc

VOCAB = 100000
MIN_N = 2
MAX_N = 3
NUM_HEADS = 4
TBL = 100000
EDIM = 64
HID = 2048
KSIZE = 3
ORDERS = MAX_N - MIN_N + 1
MEMD = ORDERS * NUM_HEADS * EDIM
BATCH = 2
SEQ = 4096
BS = BATCH * SEQ

_rng = np.random.RandomState(1234)
_HASH_MULT = ((_rng.randint(1, 2**31 - 1, size=(NUM_HEADS, MAX_N)) * 2 + 1)
              % (2**32)).astype(np.uint32)

NW = 32
CHUNK = BS // NW
GSUB = 128
NSUB = CHUNK // GSUB


def _sc_gather_mem(tables_flat, idt, idm1, idm2):
    mesh = plsc.VectorSubcoreMesh(core_axis_name="c", subcore_axis_name="s")

    @functools.partial(
        pl.kernel,
        mesh=mesh,
        compiler_params=pltpu.CompilerParams(use_tc_tiling_on_sc=False),
        out_type=jax.ShapeDtypeStruct((ORDERS * NUM_HEADS, BS, EDIM), jnp.float32),
        scratch_types=[
            pltpu.VMEM((CHUNK,), jnp.int32),
            pltpu.VMEM((CHUNK,), jnp.int32),
            pltpu.VMEM((CHUNK,), jnp.int32),
            pltpu.VMEM((NSUB, GSUB), jnp.int32),
            pltpu.VMEM((CHUNK, EDIM), jnp.float32),
            pltpu.SemaphoreType.DMA,
        ],
    )
    def gk(tbl_hbm, idt_hbm, idm1_hbm, idm2_hbm, out_hbm,
           idt_v, idm1_v, idm2_v, idx_v, rows_v, sem):
        wid = lax.axis_index("s") * 2 + lax.axis_index("c")
        base = wid * CHUNK
        pltpu.sync_copy(idt_hbm.at[pl.ds(base, CHUNK)], idt_v)
        pltpu.sync_copy(idm1_hbm.at[pl.ds(base, CHUNK)], idm1_v)
        pltpu.sync_copy(idm2_hbm.at[pl.ds(base, CHUNK)], idm2_v)

        for o, n in enumerate(range(MIN_N, MAX_N + 1)):
            for h in range(NUM_HEADS):
                kk = o * NUM_HEADS + h
                m = _HASH_MULT[h]
                for v in range(CHUNK // 16):
                    sl = pl.ds(v * 16, 16)
                    t0 = plsc.bitcast(idt_v[sl], jnp.uint32)
                    t1 = plsc.bitcast(idm1_v[sl], jnp.uint32)
                    if n == 2:
                        acc = t1 * jnp.uint32(m[0]) + t0 * jnp.uint32(m[1])
                    else:
                        t2 = plsc.bitcast(idm2_v[sl], jnp.uint32)
                        acc = (t2 * jnp.uint32(m[0]) + t1 * jnp.uint32(m[1])
                               + t0 * jnp.uint32(m[2]))
                    acc = acc ^ (acc >> jnp.uint32(16))
                    acc = acc % jnp.uint32(TBL)
                    row = plsc.bitcast(acc, jnp.int32)
                    idx_v[v // (GSUB // 16), pl.ds((v % (GSUB // 16)) * 16, 16)] = row
                cps = [
                    pltpu.async_copy(
                        tbl_hbm.at[kk].at[idx_v.at[c]],
                        rows_v.at[pl.ds(c * GSUB, GSUB)],
                        sem,
                    )
                    for c in range(NSUB)
                ]
                for cp in cps:
                    cp.wait()
                pltpu.sync_copy(rows_v, out_hbm.at[kk, pl.ds(base, CHUNK)])

    return gk(tables_flat, idt, idm1, idm2)


_TBLK = 512


def _tc_body(hs_ref, mem_ref, wg_ref, wv_ref, bg_ref, bv_ref, cw_ref, cb_ref,
             out_ref, carry_ref):
    j = pl.program_id(1)
    hs = hs_ref[0]
    mem = jnp.concatenate(
        [mem_ref[kk] for kk in range(ORDERS * NUM_HEADS)], axis=1
    )
    memh = mem.astype(jnp.bfloat16)
    keyv = jnp.dot(hs.astype(jnp.bfloat16), wg_ref[...],
                   preferred_element_type=jnp.float32) + bg_ref[...]
    dot = jnp.sum(keyv * mem, axis=1, keepdims=True) * (1.0 / np.sqrt(MEMD))
    alpha = 1.0 / (1.0 + jnp.exp(-dot))
    value = jnp.dot(memh, wv_ref[...],
                    preferred_element_type=jnp.float32) + bv_ref[...]
    gated = alpha * value
    prev = jnp.where(j == 0, 0.0, carry_ref[0:2])
    g_m1 = jnp.concatenate([prev[1:2], gated[:-1]], axis=0)
    g_m2 = jnp.concatenate([prev[0:2], gated[:-2]], axis=0)
    fused = (g_m2 * cw_ref[0:1] + g_m1 * cw_ref[1:2] + gated * cw_ref[2:3]
             + cb_ref[...])
    out_ref[0] = hs + fused
    carry_ref[0:2] = gated[_TBLK - 2:]


def _tc_dense(hs, mem3, Wg, bg, Wv, bv, conv_w, conv_b):
    grid = (BATCH, SEQ // _TBLK)
    return pl.pallas_call(
        _tc_body,
        grid=grid,
        in_specs=[
            pl.BlockSpec((1, _TBLK, HID), lambda b, j: (b, j, 0)),
            pl.BlockSpec((ORDERS * NUM_HEADS, _TBLK, EDIM),
                         lambda b, j: (0, b * (SEQ // _TBLK) + j, 0)),
            pl.BlockSpec((HID, MEMD), lambda b, j: (0, 0)),
            pl.BlockSpec((MEMD, HID), lambda b, j: (0, 0)),
            pl.BlockSpec((1, MEMD), lambda b, j: (0, 0)),
            pl.BlockSpec((1, HID), lambda b, j: (0, 0)),
            pl.BlockSpec((KSIZE, HID), lambda b, j: (0, 0)),
            pl.BlockSpec((1, HID), lambda b, j: (0, 0)),
        ],
        out_specs=pl.BlockSpec((1, _TBLK, HID), lambda b, j: (b, j, 0)),
        out_shape=jax.ShapeDtypeStruct((BATCH, SEQ, HID), jnp.float32),
        scratch_shapes=[pltpu.VMEM((8, HID), jnp.float32)],
        compiler_params=pltpu.CompilerParams(
            dimension_semantics=("arbitrary", "arbitrary"),
        ),
    )(hs, mem3, Wg.astype(jnp.bfloat16), Wv.astype(jnp.bfloat16),
      bg.reshape(1, MEMD), bv.reshape(1, HID),
      conv_w.T, conv_b.reshape(1, HID))


def kernel(hidden_states, input_ids, tables, Wg, bg, Wv, bv, conv_w, conv_b):
    ids = input_ids.astype(jnp.int32)
    idm1 = jnp.pad(ids, ((0, 0), (1, 0)))[:, :SEQ]
    idm2 = jnp.pad(ids, ((0, 0), (2, 0)))[:, :SEQ]
    mem = _sc_gather_mem(tables, ids.reshape(BS), idm1.reshape(BS),
                         idm2.reshape(BS))
    return _tc_dense(hidden_states, mem, Wg, bg, Wv, bv, conv_w, conv_b)


# speedup vs baseline: 1.1218x; 1.1204x over previous
import functools

import jax
import jax.numpy as jnp
import numpy as np
from jax import lax
from jax.experimental import pallas as pl
from jax.experimental.pallas import tpu as pltpu
from jax.experimental.pallas import tpu_sc as plsc

VOCAB = 100000
MIN_N = 2
MAX_N = 3
NUM_HEADS = 4
TBL = 100000
EDIM = 64
HID = 2048
KSIZE = 3
ORDERS = MAX_N - MIN_N + 1
MEMD = ORDERS * NUM_HEADS * EDIM
BATCH = 2
SEQ = 4096
BS = BATCH * SEQ

_rng = np.random.RandomState(1234)
_HASH_MULT = ((_rng.randint(1, 2**31 - 1, size=(NUM_HEADS, MAX_N)) * 2 + 1)
              % (2**32)).astype(np.uint32)

NW = 32
CHUNK = BS // NW
GSUB = 128
NSUB = CHUNK // GSUB


def _sc_gather_mem(tables_flat, idt, idm1, idm2):
    mesh = plsc.VectorSubcoreMesh(core_axis_name="c", subcore_axis_name="s")

    @functools.partial(
        pl.kernel,
        mesh=mesh,
        out_type=jax.ShapeDtypeStruct((ORDERS * NUM_HEADS, BS, 2 * EDIM),
                                      jnp.float32),
        scratch_types=[
            pltpu.VMEM((CHUNK,), jnp.int32),
            pltpu.VMEM((CHUNK,), jnp.int32),
            pltpu.VMEM((CHUNK,), jnp.int32),
            pltpu.VMEM((NSUB, GSUB), jnp.int32),
            pltpu.VMEM((CHUNK, 2 * EDIM), jnp.float32),
            pltpu.SemaphoreType.DMA,
        ],
    )
    def gk(tbl_hbm, idt_hbm, idm1_hbm, idm2_hbm, out_hbm,
           idt_v, idm1_v, idm2_v, idx_v, rows_v, sem):
        wid = lax.axis_index("s") * 2 + lax.axis_index("c")
        base = wid * CHUNK
        pltpu.sync_copy(idt_hbm.at[pl.ds(base, CHUNK)], idt_v)
        pltpu.sync_copy(idm1_hbm.at[pl.ds(base, CHUNK)], idm1_v)
        pltpu.sync_copy(idm2_hbm.at[pl.ds(base, CHUNK)], idm2_v)

        for o, n in enumerate(range(MIN_N, MAX_N + 1)):
            for h in range(NUM_HEADS):
                kk = o * NUM_HEADS + h
                m = _HASH_MULT[h]
                for v in range(CHUNK // 16):
                    sl = pl.ds(v * 16, 16)
                    t0 = plsc.bitcast(idt_v[sl], jnp.uint32)
                    t1 = plsc.bitcast(idm1_v[sl], jnp.uint32)
                    if n == 2:
                        acc = t1 * jnp.uint32(m[0]) + t0 * jnp.uint32(m[1])
                    else:
                        t2 = plsc.bitcast(idm2_v[sl], jnp.uint32)
                        acc = (t2 * jnp.uint32(m[0]) + t1 * jnp.uint32(m[1])
                               + t0 * jnp.uint32(m[2]))
                    acc = acc ^ (acc >> jnp.uint32(16))
                    acc = acc % jnp.uint32(TBL)
                    row = plsc.bitcast(acc, jnp.int32) + jnp.int32(kk * TBL)
                    idx_v[v // (GSUB // 16), pl.ds((v % (GSUB // 16)) * 16, 16)] = row
                cps = [
                    pltpu.async_copy(
                        tbl_hbm.at[idx_v.at[c]],
                        rows_v.at[pl.ds(c * GSUB, GSUB)],
                        sem,
                    )
                    for c in range(NSUB)
                ]
                for cp in cps:
                    cp.wait()
                pltpu.sync_copy(rows_v, out_hbm.at[kk, pl.ds(base, CHUNK)])

    return gk(tables_flat, idt, idm1, idm2)


_TBLK = 512


def _tc_body(hs_ref, mem_ref, wg_ref, wv_ref, bg_ref, bv_ref, cw_ref, cb_ref,
             out_ref, carry_ref):
    j = pl.program_id(1)
    hs = hs_ref[0]
    mem = jnp.concatenate(
        [mem_ref[kk][:, :EDIM] for kk in range(ORDERS * NUM_HEADS)], axis=1
    )
    memh = mem.astype(jnp.bfloat16)
    keyv = jnp.dot(hs.astype(jnp.bfloat16), wg_ref[...],
                   preferred_element_type=jnp.float32) + bg_ref[...]
    dot = jnp.sum(keyv * mem, axis=1, keepdims=True) * (1.0 / np.sqrt(MEMD))
    alpha = 1.0 / (1.0 + jnp.exp(-dot))
    value = jnp.dot(memh, wv_ref[...],
                    preferred_element_type=jnp.float32) + bv_ref[...]
    gated = alpha * value
    prev = jnp.where(j == 0, 0.0, carry_ref[0:2])
    g_m1 = jnp.concatenate([prev[1:2], gated[:-1]], axis=0)
    g_m2 = jnp.concatenate([prev[0:2], gated[:-2]], axis=0)
    fused = (g_m2 * cw_ref[0:1] + g_m1 * cw_ref[1:2] + gated * cw_ref[2:3]
             + cb_ref[...])
    out_ref[0] = hs + fused
    carry_ref[0:2] = gated[_TBLK - 2:]


def _tc_dense(hs, mem3, Wg, bg, Wv, bv, conv_w, conv_b):
    grid = (BATCH, SEQ // _TBLK)
    return pl.pallas_call(
        _tc_body,
        grid=grid,
        in_specs=[
            pl.BlockSpec((1, _TBLK, HID), lambda b, j: (b, j, 0)),
            pl.BlockSpec((ORDERS * NUM_HEADS, _TBLK, 2 * EDIM),
                         lambda b, j: (0, b * (SEQ // _TBLK) + j, 0)),
            pl.BlockSpec((HID, MEMD), lambda b, j: (0, 0)),
            pl.BlockSpec((MEMD, HID), lambda b, j: (0, 0)),
            pl.BlockSpec((1, MEMD), lambda b, j: (0, 0)),
            pl.BlockSpec((1, HID), lambda b, j: (0, 0)),
            pl.BlockSpec((KSIZE, HID), lambda b, j: (0, 0)),
            pl.BlockSpec((1, HID), lambda b, j: (0, 0)),
        ],
        out_specs=pl.BlockSpec((1, _TBLK, HID), lambda b, j: (b, j, 0)),
        out_shape=jax.ShapeDtypeStruct((BATCH, SEQ, HID), jnp.float32),
        scratch_shapes=[pltpu.VMEM((8, HID), jnp.float32)],
        compiler_params=pltpu.CompilerParams(
            dimension_semantics=("arbitrary", "arbitrary"),
        ),
    )(hs, mem3, Wg.astype(jnp.bfloat16), Wv.astype(jnp.bfloat16),
      bg.reshape(1, MEMD), bv.reshape(1, HID),
      conv_w.T, conv_b.reshape(1, HID))


def kernel(hidden_states, input_ids, tables, Wg, bg, Wv, bv, conv_w, conv_b):
    ids = input_ids.astype(jnp.int32)
    idm1 = jnp.pad(ids, ((0, 0), (1, 0)))[:, :SEQ]
    idm2 = jnp.pad(ids, ((0, 0), (2, 0)))[:, :SEQ]
    tables_pad = jnp.pad(tables.reshape(ORDERS * NUM_HEADS * TBL, EDIM),
                         ((0, 0), (0, EDIM)))
    mem = _sc_gather_mem(tables_pad, ids.reshape(BS), idm1.reshape(BS),
                         idm2.reshape(BS))
    return _tc_dense(hidden_states, mem, Wg, bg, Wv, bv, conv_w, conv_b)
